# baseline (speedup 1.0000x reference)
import functools

import jax
import jax.numpy as jnp
from jax import lax
from jax.experimental import pallas as pl
from jax.experimental.pallas import tpu as pltpu
from jax.experimental.pallas import tpu_sc as plsc

D_MODEL = 64
NC, NS = 2, 16
NW = NC * NS
CHUNK = 128
NBUF = 8


@functools.partial(jax.jit, static_argnums=(2, 3))
def _embedding_gather(x_resh, w, n_chunks, total_rows):
    mesh = plsc.VectorSubcoreMesh(core_axis_name="c", subcore_axis_name="s")

    scratch = [
        pltpu.VMEM((n_chunks, CHUNK), jnp.int32),
        pltpu.VMEM((NBUF, CHUNK, D_MODEL), jnp.float32),
    ] + [pltpu.SemaphoreType.DMA] * (2 * NBUF)

    @functools.partial(
        pl.kernel,
        out_type=jax.ShapeDtypeStruct((total_rows, D_MODEL), jnp.float32),
        mesh=mesh,
        scratch_types=scratch,
        compiler_params=pltpu.CompilerParams(use_tc_tiling_on_sc=False),
    )
    def k(x_hbm, w_hbm, out_hbm, idx_v, rows, *sems):
        gsem = sems[:NBUF]
        wsem = sems[NBUF:]
        wid = lax.axis_index("s") * NC + lax.axis_index("c")
        row_base = wid * (n_chunks * CHUNK)

        pltpu.sync_copy(x_hbm.at[wid], idx_v)

        def start_gather(j, s):
            pltpu.async_copy(w_hbm.at[idx_v.at[j]], rows.at[s], gsem[s])

        def wait_gather(s):
            pltpu.make_async_copy(w_hbm.at[idx_v.at[0]], rows.at[s], gsem[s]).wait()

        def start_write(j, s):
            dst = out_hbm.at[pl.ds(row_base + j * CHUNK, CHUNK)]
            pltpu.async_copy(rows.at[s], dst, wsem[s])

        def wait_write(s):
            dst = out_hbm.at[pl.ds(row_base, CHUNK)]
            pltpu.make_async_copy(rows.at[s], dst, wsem[s]).wait()

        for s in range(NBUF):
            start_gather(s, s)

        @pl.loop(0, n_chunks // NBUF)
        def _(t):
            jo = t * NBUF
            for s in range(NBUF):
                wait_gather(s)
                start_write(jo + s, s)
            for s in range(NBUF):
                nxt = jo + NBUF + s

                @pl.when(nxt < n_chunks)
                def _():
                    wait_write(s)
                    start_gather(nxt, s)

        for s in range(NBUF):
            wait_write(s)

    return k(x_resh, w)


def kernel(x, w):
    B, S = x.shape
    total = B * S
    per_w = total // NW
    n_chunks = per_w // CHUNK
    x_resh = x.reshape(NW, n_chunks, CHUNK).astype(jnp.int32)
    out = _embedding_gather(x_resh, w, n_chunks, total)
    return out.reshape(B, S, D_MODEL)
